# baseline (speedup 1.0000x reference)
import functools

import jax
import jax.numpy as jnp
from jax import lax
from jax.experimental import pallas as pl
from jax.experimental.pallas import tpu as pltpu
from jax.experimental.pallas import tpu_sc as plsc

B = 32
NOISE_STD = 0.05


def _t_gather_kernel(t_epl_hbm, idx_hbm, t_out, tab_s, idx_s, t_s):
    cid = lax.axis_index("c")

    @pl.when(cid == 0)
    def _():
        pltpu.sync_copy(t_epl_hbm, tab_s)
        pltpu.sync_copy(idx_hbm, idx_s)
        for i in range(B):
            t_s[i] = tab_s[idx_s[i]]
        pltpu.sync_copy(t_s, t_out)


def _gather_t(t_epl, random_indices):
    mesh = plsc.ScalarSubcoreMesh(axis_name="c", num_cores=1)
    kern = functools.partial(
        pl.kernel,
        mesh=mesh,
        out_type=jax.ShapeDtypeStruct((B,), jnp.int32),
        scratch_types=[
            pltpu.SMEM((64,), jnp.int32),
            pltpu.SMEM((B,), jnp.int32),
            pltpu.SMEM((B,), jnp.int32),
        ],
    )(_t_gather_kernel)
    return kern(t_epl, random_indices)


def _combine_kernel(idx_ref, t_epl_ref, atab_ref, btab_ref, x_ref, n_ref, o_ref):
    i = pl.program_id(0)
    t = t_epl_ref[idx_ref[i]]
    a = atab_ref[t]
    b = btab_ref[t] * NOISE_STD
    o_ref[...] = a * x_ref[...] + b * n_ref[...]


def _combine(idx, t_epl, atab, btab, x, n):
    _, C, H, W = x.shape
    return pl.pallas_call(
        _combine_kernel,
        grid=(B, C),
        in_specs=[
            pl.BlockSpec(memory_space=pltpu.SMEM),
            pl.BlockSpec(memory_space=pltpu.SMEM),
            pl.BlockSpec(memory_space=pltpu.SMEM),
            pl.BlockSpec(memory_space=pltpu.SMEM),
            pl.BlockSpec((1, 1, H, W), lambda i, c: (i, c, 0, 0)),
            pl.BlockSpec((1, 1, H, W), lambda i, c: (i, c, 0, 0)),
        ],
        out_specs=pl.BlockSpec((1, 1, H, W), lambda i, c: (i, c, 0, 0)),
        out_shape=jax.ShapeDtypeStruct(x.shape, jnp.float32),
    )(idx, t_epl, atab, btab, x, n)


def kernel(x_0, alphas_bar_sqrt, one_minus_alphas_bar_sqrt, t_epl, random_indices, noise):
    t = _gather_t(t_epl, random_indices)
    out = _combine(random_indices, t_epl, alphas_bar_sqrt,
                   one_minus_alphas_bar_sqrt, x_0, noise)
    return (out, t.reshape(-1, 1))


# speedup vs baseline: 1.1951x; 1.1951x over previous
import functools

import jax
import jax.numpy as jnp
from jax import lax
from jax.experimental import pallas as pl
from jax.experimental.pallas import tpu as pltpu
from jax.experimental.pallas import tpu_sc as plsc

B = 32
NOISE_STD = 0.05


def _t_gather_kernel(t_epl_hbm, idx_hbm, t_out, tab_s, idx_s, t_s):
    cid = lax.axis_index("c")

    @pl.when(cid == 0)
    def _():
        pltpu.sync_copy(t_epl_hbm, tab_s)
        pltpu.sync_copy(idx_hbm, idx_s)
        for i in range(B):
            t_s[i] = tab_s[idx_s[i]]
        pltpu.sync_copy(t_s, t_out)


def _gather_t(t_epl, random_indices):
    mesh = plsc.ScalarSubcoreMesh(axis_name="c", num_cores=1)
    kern = functools.partial(
        pl.kernel,
        mesh=mesh,
        out_type=jax.ShapeDtypeStruct((B,), jnp.int32),
        scratch_types=[
            pltpu.SMEM((64,), jnp.int32),
            pltpu.SMEM((B,), jnp.int32),
            pltpu.SMEM((B,), jnp.int32),
        ],
    )(_t_gather_kernel)
    return kern(t_epl, random_indices)


def _combine_kernel(idx_ref, t_epl_ref, atab_ref, btab_ref, x_ref, n_ref, o_ref):
    i = pl.program_id(0)
    t0 = t_epl_ref[idx_ref[2 * i]]
    t1 = t_epl_ref[idx_ref[2 * i + 1]]
    for j, t in enumerate((t0, t1)):
        a = atab_ref[t]
        b = btab_ref[t] * NOISE_STD
        o_ref[j] = a * x_ref[j] + b * n_ref[j]


def _combine(idx, t_epl, atab, btab, x, n):
    _, C, H, W = x.shape
    return pl.pallas_call(
        _combine_kernel,
        grid=(B // 2,),
        in_specs=[
            pl.BlockSpec(memory_space=pltpu.SMEM),
            pl.BlockSpec(memory_space=pltpu.SMEM),
            pl.BlockSpec(memory_space=pltpu.SMEM),
            pl.BlockSpec(memory_space=pltpu.SMEM),
            pl.BlockSpec((2, C, H, W), lambda i: (i, 0, 0, 0)),
            pl.BlockSpec((2, C, H, W), lambda i: (i, 0, 0, 0)),
        ],
        out_specs=pl.BlockSpec((2, C, H, W), lambda i: (i, 0, 0, 0)),
        out_shape=jax.ShapeDtypeStruct(x.shape, jnp.float32),
    )(idx, t_epl, atab, btab, x, n)


def kernel(x_0, alphas_bar_sqrt, one_minus_alphas_bar_sqrt, t_epl, random_indices, noise):
    t = _gather_t(t_epl, random_indices)
    out = _combine(random_indices, t_epl, alphas_bar_sqrt,
                   one_minus_alphas_bar_sqrt, x_0, noise)
    return (out, t.reshape(-1, 1))
